# Initial kernel scaffold; baseline (speedup 1.0000x reference)
import jax
import jax.numpy as jnp
from jax.experimental import pallas as pl


def kernel(Yhat):
    raise NotImplementedError("write your pallas kernel here")


# speedup vs baseline: 12.1751x; 12.1751x over previous
import functools

import jax
import jax.numpy as jnp
from jax.experimental import pallas as pl
from jax.experimental.pallas import tpu as pltpu

_B = 64
_N = 32768
_K = 256
_ROWS_PER_BLOCK = 8

def _topk_mask_kernel(y_ref, o_ref, ukey_ref, ieq_ref):
    int_min = jnp.int32(-(2 ** 31))
    b = jax.lax.bitcast_convert_type(y_ref[...], jnp.int32)
    b = jnp.where(b == int_min, jnp.int32(0), b)
    m = jax.lax.shift_right_arithmetic(b, 31)
    u = jax.lax.bitcast_convert_type(b ^ (m | int_min), jnp.uint32)
    ukey_ref[...] = u

    rows = y_ref.shape[0]
    one = jnp.uint32(1)
    kb = jnp.int32(_K)

    def val_step(i, prefix):
        bit = jnp.uint32(31) - i.astype(jnp.uint32)
        cand = prefix | (one << bit)
        cnt = jnp.sum((ukey_ref[...] >= cand).astype(jnp.int32), axis=1,
                      keepdims=True)
        return jnp.where(cnt >= kb, cand, prefix)

    tstar = jax.lax.fori_loop(0, 32, val_step,
                              jnp.zeros((rows, 1), jnp.uint32))

    cnt_gt = jnp.sum((ukey_ref[...] > tstar).astype(jnp.int32), axis=1,
                     keepdims=True)
    needed = kb - cnt_gt

    idx = jax.lax.broadcasted_iota(jnp.int32, (rows, _N), 1)
    ieq_ref[...] = jnp.where(ukey_ref[...] == tstar, idx,
                             jnp.int32(2 ** 31 - 1))

    def idx_step(i, j):
        bit = jnp.int32(14) - i
        cand = j | (jnp.int32(1) << bit)
        cnt = jnp.sum((ieq_ref[...] <= cand).astype(jnp.int32), axis=1,
                      keepdims=True)
        return jnp.where(cnt <= needed, cand, j)

    jstar = jax.lax.fori_loop(0, 15, idx_step, jnp.zeros((rows, 1), jnp.int32))

    o_ref[...] = jnp.where((ukey_ref[...] > tstar) | (ieq_ref[...] <= jstar),
                           jnp.float32(1.0), jnp.float32(0.0))


@jax.jit
def kernel(Yhat):
    grid = _B // _ROWS_PER_BLOCK
    return pl.pallas_call(
        _topk_mask_kernel,
        grid=(grid,),
        in_specs=[pl.BlockSpec((_ROWS_PER_BLOCK, _N), lambda i: (i, 0))],
        out_specs=pl.BlockSpec((_ROWS_PER_BLOCK, _N), lambda i: (i, 0)),
        out_shape=jax.ShapeDtypeStruct((_B, _N), jnp.float32),
        scratch_shapes=[
            pltpu.VMEM((_ROWS_PER_BLOCK, _N), jnp.uint32),
            pltpu.VMEM((_ROWS_PER_BLOCK, _N), jnp.int32),
        ],
    )(Yhat)
